# Initial kernel scaffold; baseline (speedup 1.0000x reference)
import jax
import jax.numpy as jnp
from jax.experimental import pallas as pl


def kernel(coord, params):
    raise NotImplementedError("write your pallas kernel here")


# speedup vs baseline: 3.7595x; 3.7595x over previous
import functools

import jax
import jax.numpy as jnp
from jax import lax
from jax.experimental import pallas as pl
from jax.experimental.pallas import tpu as pltpu
from jax.experimental.pallas import tpu_sc as plsc

N = 2_000_000
C = 32
H = 1024
W = 1024
HW = H * W
L = 16
NC = 2
NS = 16
NW = NC * NS
B = 128
NCH = N // B


def _make_sc_kernel():
    mesh = plsc.VectorSubcoreMesh(core_axis_name="c", subcore_axis_name="s")

    @functools.partial(
        pl.kernel,
        mesh=mesh,
        compiler_params=pltpu.CompilerParams(use_tc_tiling_on_sc=False),
        out_type=jax.ShapeDtypeStruct((N, C), jnp.float32),
        scratch_types=[
            pltpu.VMEM((B,), jnp.float32),
            pltpu.VMEM((B,), jnp.float32),
            pltpu.VMEM((B,), jnp.int32),
            pltpu.VMEM((B,), jnp.int32),
            pltpu.VMEM((B,), jnp.int32),
            pltpu.VMEM((B,), jnp.int32),
            pltpu.VMEM((B,), jnp.float32),
            pltpu.VMEM((B,), jnp.float32),
            pltpu.VMEM((B,), jnp.float32),
            pltpu.VMEM((B,), jnp.float32),
            pltpu.VMEM((B, C), jnp.float32),
            pltpu.VMEM((B, C), jnp.float32),
            pltpu.VMEM((B, C), jnp.float32),
            pltpu.VMEM((B, C), jnp.float32),
            pltpu.VMEM((B, C), jnp.float32),
            pltpu.SemaphoreType.DMA,
        ],
    )
    def grid_sample_sc(x_hbm, y_hbm, table_hbm, out_hbm,
                       x_v, y_v, i00, i01, i10, i11,
                       w00v, w01v, w10v, w11v,
                       r00, r01, r10, r11, out_v, sem):
        wid = lax.axis_index("s") * NC + lax.axis_index("c")
        niter = (NCH - wid + NW - 1) // NW

        def chunk_body(it, carry):
            g = wid + it * NW
            base = g * B
            pltpu.sync_copy(x_hbm.at[pl.ds(base, B)], x_v)
            pltpu.sync_copy(y_hbm.at[pl.ds(base, B)], y_v)

            for j in range(B // L):
                sj = pl.ds(j * L, L)
                x = x_v[sj]
                y = y_v[sj]
                ix = (x + 1.0) * ((W - 1) * 0.5)
                iy = (y + 1.0) * ((H - 1) * 0.5)
                ix0 = jnp.maximum(ix.astype(jnp.int32), 0)
                iy0 = jnp.maximum(iy.astype(jnp.int32), 0)
                ix0 = jnp.minimum(ix0, W - 1)
                iy0 = jnp.minimum(iy0, H - 1)
                wx1 = ix - ix0.astype(jnp.float32)
                wy1 = iy - iy0.astype(jnp.float32)
                wx0 = 1.0 - wx1
                wy0 = 1.0 - wy1
                ix1 = jnp.minimum(ix0 + 1, W - 1)
                iy1 = jnp.minimum(iy0 + 1, H - 1)
                row0 = iy0 * W
                row1 = iy1 * W
                sl = pl.ds(j * L, L)
                i00[sl] = row0 + ix0
                i01[sl] = row0 + ix1
                i10[sl] = row1 + ix0
                i11[sl] = row1 + ix1
                w00v[sl] = wy0 * wx0
                w01v[sl] = wy0 * wx1
                w10v[sl] = wy1 * wx0
                w11v[sl] = wy1 * wx1

            cps = [
                pltpu.async_copy(table_hbm.at[i00], r00, sem),
                pltpu.async_copy(table_hbm.at[i01], r01, sem),
                pltpu.async_copy(table_hbm.at[i10], r10, sem),
                pltpu.async_copy(table_hbm.at[i11], r11, sem),
            ]
            for cp in cps:
                cp.wait()

            def blend_group(j, c2):
                jb = j * L
                wa = w00v[pl.ds(jb, L)]
                wb = w01v[pl.ds(jb, L)]
                wc = w10v[pl.ds(jb, L)]
                wd = w11v[pl.ds(jb, L)]
                for i in range(L):
                    b = jb + i
                    for ch in range(C // L):
                        s = pl.ds(ch * L, L)
                        acc = (r00[b, s] * wa[i] + r01[b, s] * wb[i]
                               + r10[b, s] * wc[i] + r11[b, s] * wd[i])
                        out_v[b, s] = acc
                return c2

            lax.fori_loop(0, B // L, blend_group, 0)
            pltpu.sync_copy(out_v, out_hbm.at[pl.ds(base, B)])
            return carry

        lax.fori_loop(0, niter, chunk_body, 0)

    return grid_sample_sc


_GRID_SAMPLE_SC = _make_sc_kernel()


def kernel(coord, params):
    table = params[0].transpose(1, 2, 0).reshape(HW, C)
    return _GRID_SAMPLE_SC(coord[:, 0], coord[:, 1], table)


# speedup vs baseline: 4.3580x; 1.1592x over previous
import functools

import jax
import jax.numpy as jnp
from jax import lax
from jax.experimental import pallas as pl
from jax.experimental.pallas import tpu as pltpu
from jax.experimental.pallas import tpu_sc as plsc

N = 2_000_000
C = 32
H = 1024
W = 1024
HW = H * W
L = 16
NC = 2
NS = 16
NW = NC * NS
B = 128
NCH = N // B


def _make_sc_kernel():
    mesh = plsc.VectorSubcoreMesh(core_axis_name="c", subcore_axis_name="s")

    @functools.partial(
        pl.kernel,
        mesh=mesh,
        compiler_params=pltpu.CompilerParams(use_tc_tiling_on_sc=False),
        out_type=jax.ShapeDtypeStruct((N, C), jnp.float32),
        scratch_types=[
            pltpu.VMEM((2, 2, B), jnp.float32),
            pltpu.VMEM((2, 4, B), jnp.int32),
            pltpu.VMEM((2, 4, B), jnp.float32),
            pltpu.VMEM((2, 4, B, C), jnp.float32),
            pltpu.VMEM((2, B, C), jnp.float32),
            pltpu.SemaphoreType.DMA((2,)),
            pltpu.SemaphoreType.DMA((2,)),
            pltpu.SemaphoreType.DMA((2,)),
        ],
    )
    def grid_sample_sc(x_hbm, y_hbm, table_hbm, out_hbm,
                       xy_v, i_v, w_v, r_v, o_v, csem, gsem, osem):
        wid = lax.axis_index("s") * NC + lax.axis_index("c")
        niter = (NCH - wid + NW - 1) // NW

        def fire_coords(it):
            p = it % 2
            base = (wid + it * NW) * B
            pltpu.async_copy(x_hbm.at[pl.ds(base, B)], xy_v.at[p, 0], csem.at[p])
            pltpu.async_copy(y_hbm.at[pl.ds(base, B)], xy_v.at[p, 1], csem.at[p])

        def stage(it):
            p = it % 2
            pltpu.make_async_copy(
                x_hbm.at[pl.ds(0, B)], xy_v.at[p, 0], csem.at[p]).wait()
            pltpu.make_async_copy(
                y_hbm.at[pl.ds(0, B)], xy_v.at[p, 1], csem.at[p]).wait()
            for j in range(B // L):
                sj = pl.ds(j * L, L)
                x = xy_v[p, 0, sj]
                y = xy_v[p, 1, sj]
                ix = (x + 1.0) * ((W - 1) * 0.5)
                iy = (y + 1.0) * ((H - 1) * 0.5)
                ix0 = jnp.minimum(jnp.maximum(ix.astype(jnp.int32), 0), W - 1)
                iy0 = jnp.minimum(jnp.maximum(iy.astype(jnp.int32), 0), H - 1)
                wx1 = ix - ix0.astype(jnp.float32)
                wy1 = iy - iy0.astype(jnp.float32)
                wx0 = 1.0 - wx1
                wy0 = 1.0 - wy1
                ix1 = jnp.minimum(ix0 + 1, W - 1)
                iy1 = jnp.minimum(iy0 + 1, H - 1)
                row0 = iy0 * W
                row1 = iy1 * W
                i_v[p, 0, sj] = row0 + ix0
                i_v[p, 1, sj] = row0 + ix1
                i_v[p, 2, sj] = row1 + ix0
                i_v[p, 3, sj] = row1 + ix1
                w_v[p, 0, sj] = wy0 * wx0
                w_v[p, 1, sj] = wy0 * wx1
                w_v[p, 2, sj] = wy1 * wx0
                w_v[p, 3, sj] = wy1 * wx1
            for k in range(4):
                pltpu.async_copy(
                    table_hbm.at[i_v.at[p, k]], r_v.at[p, k], gsem.at[p])

        def consume(it):
            p = it % 2
            base = (wid + it * NW) * B
            for k in range(4):
                pltpu.make_async_copy(
                    table_hbm.at[i_v.at[p, k]], r_v.at[p, k], gsem.at[p]).wait()

            @pl.when(it >= 2)
            def _():
                pltpu.make_async_copy(
                    o_v.at[p], out_hbm.at[pl.ds(0, B)], osem.at[p]).wait()

            def blend_group(j, c2):
                jb = j * L
                wa = w_v[p, 0, pl.ds(jb, L)]
                wb = w_v[p, 1, pl.ds(jb, L)]
                wc = w_v[p, 2, pl.ds(jb, L)]
                wd = w_v[p, 3, pl.ds(jb, L)]
                for i in range(L):
                    b = jb + i
                    for ch in range(C // L):
                        s = pl.ds(ch * L, L)
                        acc = (r_v[p, 0, b, s] * wa[i] + r_v[p, 1, b, s] * wb[i]
                               + r_v[p, 2, b, s] * wc[i] + r_v[p, 3, b, s] * wd[i])
                        o_v[p, b, s] = acc
                return c2

            lax.fori_loop(0, B // L, blend_group, 0)
            pltpu.async_copy(o_v.at[p], out_hbm.at[pl.ds(base, B)], osem.at[p])

        fire_coords(0)
        fire_coords(1)
        stage(0)

        def body(it, carry):
            @pl.when(it + 2 < niter)
            def _():
                fire_coords(it + 2)

            @pl.when(it + 1 < niter)
            def _():
                stage(it + 1)

            consume(it)
            return carry

        lax.fori_loop(0, niter, body, 0)
        for p in range(2):
            pltpu.make_async_copy(
                o_v.at[p], out_hbm.at[pl.ds(0, B)], osem.at[p]).wait()

    return grid_sample_sc


_GRID_SAMPLE_SC = _make_sc_kernel()


def kernel(coord, params):
    table = params[0].transpose(1, 2, 0).reshape(HW, C)
    return _GRID_SAMPLE_SC(coord[:, 0], coord[:, 1], table)
